# baseline (speedup 1.0000x reference)
import functools

import jax
import jax.numpy as jnp
from jax import lax
from jax.experimental import pallas as pl
from jax.experimental.pallas import tpu as pltpu
from jax.experimental.pallas import tpu_sc as plsc

VOCAB = 100000
HID = 128
MAXPOS = 512
VDIM = 2048
B = 1024
S = 200
NB = 36
EPS = 1e-12

L = 16
NC = 2
NS = 16
NW = NC * NS
TOTAL = B * S
PER_W = TOTAL // NW
CHUNK = 128
NCHUNK = PER_W // CHUNK
NV = HID // L


_GDN = lax.GatherDimensionNumbers(
    offset_dims=(), collapsed_slice_dims=(0,), start_index_map=(0,))


def _lane_shuffle(x, idx):
    return lax.gather(x, idx[:, None], _GDN, (1,),
                      mode=lax.GatherScatterMode.PROMISE_IN_BOUNDS)


def _hsum_splat(x, lanes):
    for k in (1, 2, 4, 8):
        x = x + _lane_shuffle(x, lanes ^ k)
    return x


def _rsqrt_sc(x):
    i = lax.bitcast_convert_type(x, jnp.int32)
    i = jnp.int32(0x5F3759DF) - (i >> 1)
    y = lax.bitcast_convert_type(i, jnp.float32)
    for _ in range(3):
        y = y * (1.5 - 0.5 * x * y * y)
    return y


def _text_sc(tid, pid, tt, word_emb, pos_emb, ln_g, ln_b):
    mesh = plsc.VectorSubcoreMesh(core_axis_name="c", subcore_axis_name="s")

    @functools.partial(
        pl.kernel,
        out_type=jax.ShapeDtypeStruct((TOTAL, HID), jnp.float32),
        mesh=mesh,
        scratch_types=[
            pltpu.VMEM((CHUNK,), jnp.int32),
            pltpu.VMEM((CHUNK,), jnp.int32),
            pltpu.VMEM((CHUNK,), jnp.int32),
            pltpu.VMEM((CHUNK, HID), jnp.float32),
            pltpu.VMEM((CHUNK, HID), jnp.float32),
            pltpu.VMEM((CHUNK, HID), jnp.float32),
            pltpu.VMEM((HID,), jnp.float32),
            pltpu.VMEM((HID,), jnp.float32),
            pltpu.SemaphoreType.DMA,
        ],
    )
    def text_kernel(tid_h, pid_h, tt_h, wtab_h, ptab_h, g_h, b_h, out_h,
                    tid_v, pid_v, tt_v, wbuf, pbuf, tbuf, g_v, b_v, sem):
        wid = lax.axis_index("s") * NC + lax.axis_index("c")
        base_w = wid * PER_W
        lanes = lax.iota(jnp.int32, L)
        pltpu.sync_copy(g_h, g_v)
        pltpu.sync_copy(b_h, b_v)

        def chunk_body(ci, carry):
            base = base_w + ci * CHUNK
            pltpu.sync_copy(tid_h.at[pl.ds(base, CHUNK)], tid_v)
            pltpu.sync_copy(pid_h.at[pl.ds(base, CHUNK)], pid_v)
            pltpu.sync_copy(tt_h.at[pl.ds(base, CHUNK)], tt_v)
            cw = pltpu.async_copy(wtab_h.at[tid_v], wbuf, sem)
            cp = pltpu.async_copy(ptab_h.at[pid_v], pbuf, sem)
            ct = pltpu.async_copy(wtab_h.at[tt_v], tbuf, sem)
            cw.wait()
            cp.wait()
            ct.wait()

            def row_body(i, rcarry):
                vs = []
                s = jnp.zeros((L,), jnp.float32)
                for j in range(NV):
                    x = (wbuf[i, pl.ds(j * L, L)]
                         + pbuf[i, pl.ds(j * L, L)]
                         + tbuf[i, pl.ds(j * L, L)])
                    vs.append(x)
                    s = s + x
                muv = _hsum_splat(s, lanes) * (1.0 / HID)
                ss = jnp.zeros((L,), jnp.float32)
                for j in range(NV):
                    d = vs[j] - muv
                    vs[j] = d
                    ss = ss + d * d
                var = _hsum_splat(ss, lanes) * (1.0 / HID)
                rv = _rsqrt_sc(var + EPS)
                for j in range(NV):
                    sl = pl.ds(j * L, L)
                    wbuf[i, sl] = vs[j] * rv * g_v[sl] + b_v[sl]
                return rcarry

            lax.fori_loop(0, CHUNK, row_body, 0, unroll=False)
            pltpu.sync_copy(wbuf, out_h.at[pl.ds(base, CHUNK)])
            return carry

        lax.fori_loop(0, NCHUNK, chunk_body, 0, unroll=False)

    return text_kernel(tid, pid, tt, word_emb, pos_emb, ln_g, ln_b)


def _ln_tc(y, g, b):
    mu = jnp.mean(y, axis=-1, keepdims=True)
    d = y - mu
    var = jnp.mean(d * d, axis=-1, keepdims=True)
    return d * lax.rsqrt(var + EPS) * g + b


def _image_tc(image_flat, img_W, img_b, iln_g, iln_b, w1row, vln_g, vln_b):
    M = B * NB
    BM = 512

    def body(x_ref, w_ref, b_ref, ig_ref, ib_ref, w1_ref, vg_ref, vb_ref, o_ref):
        y = jnp.dot(x_ref[...], w_ref[...], preferred_element_type=jnp.float32)
        y = y + b_ref[...]
        y = _ln_tc(y, ig_ref[...], ib_ref[...])
        y = y + w1_ref[...]
        o_ref[...] = _ln_tc(y, vg_ref[...], vb_ref[...])

    row_spec = pl.BlockSpec((1, HID), lambda i: (0, 0))
    return pl.pallas_call(
        body,
        grid=(M // BM,),
        in_specs=[
            pl.BlockSpec((BM, VDIM), lambda i: (i, 0)),
            pl.BlockSpec((VDIM, HID), lambda i: (0, 0)),
            row_spec, row_spec, row_spec, row_spec, row_spec, row_spec,
        ],
        out_specs=pl.BlockSpec((BM, HID), lambda i: (i, 0)),
        out_shape=jax.ShapeDtypeStruct((M, HID), jnp.float32),
    )(image_flat, img_W, img_b, iln_g, iln_b, w1row, vln_g, vln_b)


def kernel(token_ids, image_feat, token_type_ids, position_ids, word_emb,
           pos_emb, img_W, img_b, ln_g, ln_b, iln_g, iln_b, vln_g, vln_b):
    tid = token_ids.reshape(-1)
    pid = position_ids.reshape(-1)
    tt = token_type_ids.reshape(-1)
    emb = _text_sc(tid, pid, tt, word_emb, pos_emb, ln_g, ln_b)
    emb = emb.reshape(B, S, HID)

    w1row = lax.slice(word_emb, (1, 0), (2, HID))
    r = lambda a: a.reshape(1, HID)
    v = _image_tc(image_feat.reshape(B * NB, VDIM), img_W, r(img_b),
                  r(iln_g), r(iln_b), w1row, r(vln_g), r(vln_b))
    return (emb, v.reshape(B, NB, HID))
